# baseline (speedup 1.0000x reference)
import jax
import jax.numpy as jnp
from jax import lax
from jax.experimental import pallas as pl
from jax.experimental.pallas import tpu as pltpu
from jax.experimental.pallas import tpu_sc as plsc
import functools

VOCAB = 1000
DIM = 16
CHARS = 20
NC = 2
NS = 16
NW = NC * NS
BT = 128


@functools.partial(jax.jit, static_argnames=("chunk",))
def _sc_embed_max(xt, W, *, chunk):
    _, S, B = xt.shape
    nch = S // chunk

    mesh = plsc.VectorSubcoreMesh(core_axis_name="c", subcore_axis_name="s")

    @functools.partial(
        pl.kernel,
        out_type=jax.ShapeDtypeStruct((S, DIM // 8, B // BT, 8, BT),
                                      jnp.float32),
        mesh=mesh,
        scratch_types=[
            pltpu.VMEM((VOCAB, DIM), jnp.float32),
            pltpu.VMEM((CHARS, chunk, BT), jnp.int32),
            pltpu.VMEM((chunk, DIM // 8, 1, 8, BT), jnp.float32),
            pltpu.VMEM((16, 17), jnp.float32),
        ],
        compiler_params=pltpu.CompilerParams(
            needs_layout_passes=False, use_tc_tiling_on_sc=False),
    )
    def k(x_hbm, w_hbm, out_hbm, w_v, idx_v, out_v, st_v):
        wid = lax.axis_index("s") * NC + lax.axis_index("c")
        pltpu.sync_copy(w_hbm, w_v)
        col = lax.iota(jnp.int32, 16)
        b0 = wid * BT

        @pl.loop(0, nch)
        def chunk_loop(ch):
            s0 = ch * chunk
            pltpu.sync_copy(
                x_hbm.at[:, pl.ds(s0, chunk), pl.ds(b0, BT)], idx_v)

            @pl.loop(0, chunk)
            def s_loop(s):

                @pl.loop(0, BT // 16, unroll=2)
                def blk_loop(l):
                    ivs = [idx_v[c, s, pl.ds(l * 16, 16)]
                           for c in range(CHARS)]
                    for j in range(16):
                        rows = []
                        for c in range(CHARS):
                            sp = jnp.take_along_axis(
                                ivs[c], jnp.full((16,), j, jnp.int32),
                                axis=0, mode="promise_in_bounds")
                            rows.append(plsc.load_gather(w_v, [sp, col]))
                        while len(rows) > 1:
                            rows = [
                                jnp.maximum(rows[i], rows[i + 1])
                                if i + 1 < len(rows) else rows[i]
                                for i in range(0, len(rows), 2)
                            ]
                        st_v[j, pl.ds(0, 16)] = rows[0]
                    for d in range(DIM):
                        tv = plsc.load_gather(
                            st_v, [col, jnp.full((16,), d, jnp.int32)])
                        out_v[s, d // 8, 0, d % 8, pl.ds(l * 16, 16)] = tv

            pltpu.sync_copy(
                out_v,
                out_hbm.at[pl.ds(s0, chunk), :, pl.ds(wid, 1)])

    return k(xt, W)


def kernel(x, W):
    B, S, _ = x.shape
    xt = jnp.transpose(x.astype(jnp.int32), (2, 1, 0))
    o = _sc_embed_max(xt, W, chunk=10)
    out = jnp.transpose(o, (2, 4, 0, 1, 3)).reshape(B, S, DIM)
    return out


# speedup vs baseline: 1.2704x; 1.2704x over previous
import jax
import jax.numpy as jnp
from jax import lax
from jax.experimental import pallas as pl
from jax.experimental.pallas import tpu as pltpu
from jax.experimental.pallas import tpu_sc as plsc
import functools

VOCAB = 1000
DIM = 16
CHARS = 20
NC = 2
NS = 16
NW = NC * NS
BT = 128


@functools.partial(jax.jit, static_argnames=("chunk",))
def _sc_embed_max(xt, Wp, *, chunk):
    _, S, B = xt.shape
    nch = S // chunk

    mesh = plsc.VectorSubcoreMesh(core_axis_name="c", subcore_axis_name="s")

    @functools.partial(
        pl.kernel,
        out_type=jax.ShapeDtypeStruct((S, DIM // 8, B // BT, 8, BT),
                                      jnp.float32),
        mesh=mesh,
        scratch_types=[
            pltpu.VMEM((VOCAB, DIM // 2), jnp.int32),
            pltpu.VMEM((CHARS, chunk, BT), jnp.int32),
            pltpu.VMEM((chunk, DIM // 8, 1, 8, BT), jnp.float32),
            pltpu.VMEM((16, 17), jnp.float32),
        ],
        compiler_params=pltpu.CompilerParams(
            needs_layout_passes=False, use_tc_tiling_on_sc=False),
    )
    def k(x_hbm, w_hbm, out_hbm, w_v, idx_v, out_v, st_v):
        wid = lax.axis_index("s") * NC + lax.axis_index("c")
        pltpu.sync_copy(w_hbm, w_v)
        col = lax.iota(jnp.int32, 16)
        col8 = jnp.bitwise_and(col, 7)
        half = jnp.where(col < 8, 0, 1)
        cols_even = jnp.left_shift(col8, 1)
        cols_odd = cols_even + 1
        hi_mask = jnp.int32(-65536)
        b0 = wid * BT

        @pl.loop(0, nch)
        def chunk_loop(ch):
            s0 = ch * chunk
            pltpu.sync_copy(
                x_hbm.at[:, pl.ds(s0, chunk), pl.ds(b0, BT)], idx_v)

            @pl.loop(0, chunk)
            def s_loop(s):

                @pl.loop(0, BT // 16)
                def blk_loop(l):
                    ivs = [idx_v[c, s, pl.ds(l * 16, 16)]
                           for c in range(CHARS)]
                    for p in range(8):
                        patt = half + (2 * p)
                        rows = []
                        for c in range(CHARS):
                            sp = jnp.take_along_axis(
                                ivs[c], patt, axis=0,
                                mode="promise_in_bounds")
                            g = plsc.load_gather(w_v, [sp, col8])
                            rows.append(plsc.bitcast(g, jnp.bfloat16))
                        while len(rows) > 1:
                            rows = [
                                jnp.maximum(rows[i], rows[i + 1])
                                if i + 1 < len(rows) else rows[i]
                                for i in range(0, len(rows), 2)
                            ]
                        mu = plsc.bitcast(rows[0], jnp.int32)
                        lo = plsc.bitcast(
                            jnp.left_shift(mu, 16), jnp.float32)
                        hi = plsc.bitcast(
                            jnp.bitwise_and(mu, hi_mask), jnp.float32)
                        plsc.store_scatter(st_v, [patt, cols_even], lo)
                        plsc.store_scatter(st_v, [patt, cols_odd], hi)
                    for d in range(DIM):
                        tv = plsc.load_gather(
                            st_v, [col, jnp.full((16,), d, jnp.int32)])
                        out_v[s, d // 8, 0, d % 8, pl.ds(l * 16, 16)] = tv

            pltpu.sync_copy(
                out_v,
                out_hbm.at[pl.ds(s0, chunk), :, pl.ds(wid, 1)])

    return k(xt, Wp)


def kernel(x, W):
    B, S, _ = x.shape
    xt = jnp.transpose(x.astype(jnp.int32), (2, 1, 0))
    Wb = W.astype(jnp.bfloat16).reshape(VOCAB, DIM // 2, 2)
    Wp = lax.bitcast_convert_type(Wb, jnp.int32)
    o = _sc_embed_max(xt, Wp, chunk=10)
    out = jnp.transpose(o, (2, 4, 0, 1, 3)).reshape(B, S, DIM)
    return out


# speedup vs baseline: 1.2917x; 1.0168x over previous
import jax
import jax.numpy as jnp
from jax import lax
from jax.experimental import pallas as pl
from jax.experimental.pallas import tpu as pltpu
from jax.experimental.pallas import tpu_sc as plsc
import functools

VOCAB = 1000
DIM = 16
CHARS = 20
NC = 2
NS = 16
NW = NC * NS
BT = 128


@functools.partial(jax.jit, static_argnames=("chunk",))
def _sc_embed_max(xt, Wp, *, chunk):
    _, S, B = xt.shape
    nch = S // chunk

    mesh = plsc.VectorSubcoreMesh(core_axis_name="c", subcore_axis_name="s")

    @functools.partial(
        pl.kernel,
        out_type=jax.ShapeDtypeStruct((S, DIM // 8, B // BT, 8, BT),
                                      jnp.float32),
        mesh=mesh,
        scratch_types=[
            pltpu.VMEM((VOCAB, DIM), jnp.int32),
            pltpu.VMEM((CHARS, chunk, BT), jnp.int32),
            pltpu.VMEM((chunk, DIM // 8, 1, 8, BT), jnp.float32),
            pltpu.VMEM((16, 17), jnp.float32),
        ],
        compiler_params=pltpu.CompilerParams(
            needs_layout_passes=False, use_tc_tiling_on_sc=False),
    )
    def k(x_hbm, w_hbm, out_hbm, w_v, idx_v, out_v, st_v):
        wid = lax.axis_index("s") * NC + lax.axis_index("c")
        pltpu.sync_copy(w_hbm, w_v)
        col = lax.iota(jnp.int32, 16)
        col8 = jnp.bitwise_and(col, 7)
        half = jnp.where(col < 8, 0, 1)
        cols_even = jnp.left_shift(col8, 1)
        cols_odd = cols_even + 1
        gcols = cols_even + half
        hi_mask = jnp.int32(-65536)
        b0 = wid * BT

        @pl.loop(0, nch)
        def chunk_loop(ch):
            s0 = ch * chunk
            pltpu.sync_copy(
                x_hbm.at[:, pl.ds(s0, chunk), pl.ds(b0, BT)], idx_v)

            @pl.loop(0, chunk)
            def s_loop(s):

                @pl.loop(0, BT // 16)
                def blk_loop(l):
                    ivs = [idx_v[c, s, pl.ds(l * 16, 16)]
                           for c in range(CHARS)]
                    for p in range(8):
                        patt = half + (2 * p)
                        rows = []
                        for c in range(CHARS):
                            sp = jnp.take_along_axis(
                                ivs[c], patt, axis=0,
                                mode="promise_in_bounds")
                            g = plsc.load_gather(w_v, [sp, gcols])
                            rows.append(plsc.bitcast(g, jnp.bfloat16))
                        while len(rows) > 1:
                            rows = [
                                jnp.maximum(rows[i], rows[i + 1])
                                if i + 1 < len(rows) else rows[i]
                                for i in range(0, len(rows), 2)
                            ]
                        mu = plsc.bitcast(rows[0], jnp.int32)
                        lo = plsc.bitcast(
                            jnp.left_shift(mu, 16), jnp.float32)
                        hi = plsc.bitcast(
                            jnp.bitwise_and(mu, hi_mask), jnp.float32)
                        plsc.store_scatter(st_v, [patt, cols_even], lo)
                        plsc.store_scatter(st_v, [patt, cols_odd], hi)
                    for d in range(DIM):
                        tv = plsc.load_gather(
                            st_v, [col, jnp.full((16,), d, jnp.int32)])
                        out_v[s, d // 8, 0, d % 8, pl.ds(l * 16, 16)] = tv

            pltpu.sync_copy(
                out_v,
                out_hbm.at[pl.ds(s0, chunk), :, pl.ds(wid, 1)])

    return k(xt, Wp)


def kernel(x, W):
    B, S, _ = x.shape
    xt = jnp.transpose(x.astype(jnp.int32), (2, 1, 0))
    Wb = W.astype(jnp.bfloat16).reshape(VOCAB, DIM // 2, 2)
    Wp = lax.bitcast_convert_type(Wb, jnp.int32)
    o = _sc_embed_max(xt, jnp.repeat(Wp, 2, axis=1), chunk=10)
    out = jnp.transpose(o, (2, 4, 0, 1, 3)).reshape(B, S, DIM)
    return out


# speedup vs baseline: 1.3514x; 1.0462x over previous
import jax
import jax.numpy as jnp
from jax import lax
from jax.experimental import pallas as pl
from jax.experimental.pallas import tpu as pltpu
from jax.experimental.pallas import tpu_sc as plsc
import functools

VOCAB = 1000
DIM = 16
CHARS = 20
NC = 2
NS = 16
NW = NC * NS
BT = 128


@functools.partial(jax.jit, static_argnames=("chunk",))
def _sc_embed_max(xt, Wp, *, chunk):
    _, S, B = xt.shape
    nch = S // chunk

    mesh = plsc.VectorSubcoreMesh(core_axis_name="c", subcore_axis_name="s")

    @functools.partial(
        pl.kernel,
        out_type=jax.ShapeDtypeStruct((S, DIM // 8, B // BT, 8, BT),
                                      jnp.float32),
        mesh=mesh,
        scratch_types=[
            pltpu.VMEM((VOCAB, DIM), jnp.int32),
            pltpu.VMEM((2, CHARS, chunk, BT), jnp.int32),
            pltpu.VMEM((chunk, DIM // 8, 1, 8, BT), jnp.float32),
            pltpu.VMEM((16, 17), jnp.float32),
            pltpu.SemaphoreType.DMA((2,)),
        ],
        compiler_params=pltpu.CompilerParams(
            needs_layout_passes=False, use_tc_tiling_on_sc=False),
    )
    def k(x_hbm, w_hbm, out_hbm, w_v, idx_v, out_v, st_v, sem):
        wid = lax.axis_index("s") * NC + lax.axis_index("c")
        pltpu.sync_copy(w_hbm, w_v)
        col = lax.iota(jnp.int32, 16)
        col8 = jnp.bitwise_and(col, 7)
        half = jnp.where(col < 8, 0, 1)
        cols_even = jnp.left_shift(col8, 1)
        cols_odd = cols_even + 1
        gcols = cols_even + half
        hi_mask = jnp.int32(-65536)
        b0 = wid * BT

        def idx_copy(ch, sl):
            return pltpu.make_async_copy(
                x_hbm.at[:, pl.ds(ch * chunk, chunk), pl.ds(b0, BT)],
                idx_v.at[sl], sem.at[sl])

        idx_copy(0, 0).start()

        @pl.loop(0, nch)
        def chunk_loop(ch):
            s0 = ch * chunk
            sl = lax.rem(ch, 2)

            @pl.when(ch + 1 < nch)
            def _():
                idx_copy(ch + 1, 1 - sl).start()

            idx_copy(ch, sl).wait()

            @pl.loop(0, chunk)
            def s_loop(s):

                @pl.loop(0, BT // 16)
                def blk_loop(l):
                    ivs = [idx_v[sl, c, s, pl.ds(l * 16, 16)]
                           for c in range(CHARS)]
                    for p in range(8):
                        patt = half + (2 * p)
                        rows = []
                        for c in range(CHARS):
                            sp = jnp.take_along_axis(
                                ivs[c], patt, axis=0,
                                mode="promise_in_bounds")
                            g = plsc.load_gather(w_v, [sp, gcols])
                            rows.append(plsc.bitcast(g, jnp.bfloat16))
                        while len(rows) > 1:
                            rows = [
                                jnp.maximum(rows[i], rows[i + 1])
                                if i + 1 < len(rows) else rows[i]
                                for i in range(0, len(rows), 2)
                            ]
                        mu = plsc.bitcast(rows[0], jnp.int32)
                        lo = plsc.bitcast(
                            jnp.left_shift(mu, 16), jnp.float32)
                        hi = plsc.bitcast(
                            jnp.bitwise_and(mu, hi_mask), jnp.float32)
                        plsc.store_scatter(st_v, [patt, cols_even], lo)
                        plsc.store_scatter(st_v, [patt, cols_odd], hi)
                    for d in range(DIM):
                        tv = plsc.load_gather(
                            st_v, [col, jnp.full((16,), d, jnp.int32)])
                        out_v[s, d // 8, 0, d % 8, pl.ds(l * 16, 16)] = tv

            pltpu.sync_copy(
                out_v,
                out_hbm.at[pl.ds(s0, chunk), :, pl.ds(wid, 1)])

    return k(xt, Wp)


def kernel(x, W):
    B, S, _ = x.shape
    xt = jnp.transpose(x.astype(jnp.int32), (2, 1, 0))
    Wb = W.astype(jnp.bfloat16).reshape(VOCAB, DIM // 2, 2)
    Wp = lax.bitcast_convert_type(Wb, jnp.int32)
    o = _sc_embed_max(xt, jnp.repeat(Wp, 2, axis=1), chunk=10)
    out = jnp.transpose(o, (2, 4, 0, 1, 3)).reshape(B, S, DIM)
    return out
